# baseline (speedup 1.0000x reference)
import functools

import jax
import jax.numpy as jnp
from jax import lax
from jax.experimental import pallas as pl
from jax.experimental.pallas import tpu as pltpu
from jax.experimental.pallas import tpu_sc as plsc

_NC = 2
_NS = 16
_NW = _NC * _NS
_L = 16


def _node_radius_kernel(attrs_ref, azf_ref, cr_ref, out_ref, cr4_ref):
    @pl.when(pl.program_id(0) == 0)
    def _():
        z128 = lax.broadcasted_iota(jnp.int32, (128, 128), 1).astype(
            jnp.float32)
        mz = (azf_ref[...] == z128).astype(jnp.float32)
        cr4_ref[...] = 0.25 * jnp.dot(mz, cr_ref[...],
                                      preferred_element_type=jnp.float32)

    attrs = attrs_ref[...]
    kf = lax.broadcasted_iota(jnp.int32, attrs.shape, 1).astype(jnp.float32)
    m = jnp.max(attrs, axis=1, keepdims=True)
    w = jnp.where(attrs == m, 128.0 - kf, 0.0)
    idxf = 128.0 - jnp.max(w, axis=1, keepdims=True)
    oh = (kf == idxf).astype(jnp.float32)
    cr4 = cr4_ref[...]
    dn = (((0,), (1,)), ((), ()))
    rows = [
        lax.dot_general(cr4, oh[128 * r:128 * (r + 1), :], dn,
                        preferred_element_type=jnp.float32)
        for r in range(attrs.shape[0] // 128)
    ]
    out_ref[...] = jnp.concatenate(rows, axis=0)


def _soft_kernel(x_ref, r0_ref, ab_ref, y_ref):
    x = x_ref[...]
    u = x / r0_ref[...].astype(jnp.float32)
    a = ab_ref[0]
    b = ab_ref[1]
    p = jnp.exp(b * jnp.log(jnp.maximum(u, 1e-30)))
    p = jnp.where(u > 0.0, p, 0.0)
    y_ref[...] = x + 0.5 * jnp.tanh(-u - a * p) + 0.5


def _make_gather_r0(E, TBL):
    FR = 8
    C = FR * 256
    frows = E // 256
    total_chunks = frows // FR
    mesh = plsc.VectorSubcoreMesh(core_axis_name="c", subcore_axis_name="s")

    @functools.partial(
        pl.kernel,
        mesh=mesh,
        compiler_params=pltpu.CompilerParams(needs_layout_passes=False),
        out_type=jax.ShapeDtypeStruct((frows, 128), jnp.float32),
        scratch_types=[
            pltpu.VMEM((TBL,), jnp.float32),
            pltpu.VMEM((2, C), jnp.int32),
            pltpu.VMEM((2, C), jnp.int32),
            pltpu.VMEM((FR, 128), jnp.float32),
            pltpu.VMEM((FR, 128), jnp.float32),
            pltpu.SemaphoreType.DMA,
            pltpu.SemaphoreType.DMA,
            pltpu.SemaphoreType.DMA,
            pltpu.SemaphoreType.DMA,
            pltpu.SemaphoreType.DMA,
        ],
    )
    def gather_r0(rnode_hbm, eidx_hbm, out_hbm, table_v,
                  eb0, eb1, r00, r01, tsem, is0, is1, os0, os1):
        wid = lax.axis_index("s") * _NC + lax.axis_index("c")
        my_chunks = (total_chunks - wid + _NW - 1) // _NW
        ebs, r0s, isems, osems = (eb0, eb1), (r00, r01), (is0, is1), (os0, os1)

        def in_src(ci):
            chunk = wid + ci * _NW
            base = pl.multiple_of(chunk * C, 128)
            return eidx_hbm.at[:, pl.ds(base, C)]

        def out_dst(ci):
            chunk = wid + ci * _NW
            rb = pl.multiple_of(chunk * FR, 8)
            return out_hbm.at[pl.ds(rb, FR), :]

        @pl.when(my_chunks > 0)
        def _():
            pltpu.async_copy(in_src(0), eb0, is0)

        @pl.when(my_chunks > 1)
        def _():
            pltpu.async_copy(in_src(1), eb1, is1)

        pltpu.async_copy(rnode_hbm.at[pl.ds(0, TBL)], table_v, tsem).wait()

        i2 = lax.iota(jnp.int32, _L) * 2
        z16 = jnp.zeros((_L,), jnp.int32)
        o16 = jnp.ones((_L,), jnp.int32)

        def pair_body(p, carry):
            for b in range(2):
                ci = p * 2 + b
                eb, r0v, isem, osem = ebs[b], r0s[b], isems[b], osems[b]

                @pl.when(ci < my_chunks)
                def _():
                    pltpu.make_async_copy(in_src(ci), eb, isem).wait()

                    @pl.when(ci >= 2)
                    def _():
                        pltpu.make_async_copy(r0v, out_dst(ci - 2), osem).wait()

                    @plsc.parallel_loop(0, FR, unroll=2)
                    def row_body(row):
                        for j in range(8):
                            pos_e = row * 256 + j * 32 + i2
                            pos_o = pos_e + 1
                            s_e = plsc.load_gather(eb, [z16, pos_e])
                            s_o = plsc.load_gather(eb, [z16, pos_o])
                            r_e = plsc.load_gather(eb, [o16, pos_e])
                            r_o = plsc.load_gather(eb, [o16, pos_o])
                            g_e = (plsc.load_gather(table_v, [s_e]) +
                                   plsc.load_gather(table_v, [r_e]))
                            g_o = (plsc.load_gather(table_v, [s_o]) +
                                   plsc.load_gather(table_v, [r_o]))
                            pk = plsc.bitcast(
                                plsc.pack(g_e, g_o,
                                          format=plsc.PackFormat.INTERLEAVED),
                                jnp.float32)
                            r0v[row, pl.ds(j * _L, _L)] = pk

                    pltpu.async_copy(r0v, out_dst(ci), osem)

                    @pl.when(ci + 2 < my_chunks)
                    def _():
                        pltpu.async_copy(in_src(ci + 2), eb, isem)

            return carry

        lax.fori_loop(0, (my_chunks + 1) // 2, pair_body, 0)

        for b in range(2):
            @pl.when(my_chunks > b)
            def _():
                pltpu.make_async_copy(r0s[b], out_dst(b), osems[b]).wait()

    return gather_r0


def kernel(x, node_attrs, edge_index, atomic_numbers, covalent_radii, a, b):
    N, K = node_attrs.shape
    E = x.shape[0]
    BN = 10240
    n_blocks = (N + BN - 1) // BN
    N_pad = n_blocks * BN
    TBL = ((N + 127) // 128 + 7) // 8 * 8 * 128

    azf = atomic_numbers.astype(jnp.float32).reshape(K, 1)
    crp = jnp.zeros((K, 1), jnp.float32)
    crp = crp.at[: covalent_radii.shape[0], 0].set(covalent_radii)

    rnode4 = pl.pallas_call(
        _node_radius_kernel,
        grid=(n_blocks,),
        in_specs=[
            pl.BlockSpec((BN, K), lambda i: (i, 0)),
            pl.BlockSpec((K, 1), lambda i: (0, 0)),
            pl.BlockSpec((K, 1), lambda i: (0, 0)),
        ],
        out_specs=pl.BlockSpec((BN // 128, 128), lambda i: (i, 0)),
        out_shape=jax.ShapeDtypeStruct((N_pad // 128, 128), jnp.float32),
        scratch_shapes=[pltpu.VMEM((128, 1), jnp.float32)],
    )(node_attrs, azf, crp)

    r0c = _make_gather_r0(E, TBL)(rnode4.reshape(N_pad), edge_index)
    r0 = jax.lax.bitcast_convert_type(r0c, jnp.bfloat16).reshape(E // 128, 128)

    W = 128
    rows = E // W
    RB = 10000
    ab = jnp.stack([a.astype(jnp.float32), b.astype(jnp.float32)])
    y = pl.pallas_call(
        _soft_kernel,
        grid=(rows // RB,),
        in_specs=[
            pl.BlockSpec((RB, W), lambda i: (i, 0)),
            pl.BlockSpec((RB, W), lambda i: (i, 0)),
            pl.BlockSpec(memory_space=pltpu.SMEM),
        ],
        out_specs=pl.BlockSpec((RB, W), lambda i: (i, 0)),
        out_shape=jax.ShapeDtypeStruct((rows, W), jnp.float32),
    )(x.reshape(rows, W), r0, ab)
    return y.reshape(E, 1)


# speedup vs baseline: 26.8000x; 26.8000x over previous
import functools

import jax
import jax.numpy as jnp
from jax import lax
from jax.experimental import pallas as pl
from jax.experimental.pallas import tpu as pltpu
from jax.experimental.pallas import tpu_sc as plsc

_NC = 2
_NS = 16
_NW = _NC * _NS
_L = 16


def _node_radius_kernel(attrs_ref, azf_ref, cr_ref, out_ref, cr4_ref):
    @pl.when(pl.program_id(0) == 0)
    def _():
        z128 = lax.broadcasted_iota(jnp.int32, (128, 128), 1).astype(
            jnp.float32)
        mz = (azf_ref[...] == z128).astype(jnp.float32)
        cr4_ref[...] = 0.25 * jnp.dot(mz, cr_ref[...],
                                      preferred_element_type=jnp.float32)

    attrs = attrs_ref[...]
    kf = lax.broadcasted_iota(jnp.int32, attrs.shape, 1).astype(jnp.float32)
    m = jnp.max(attrs, axis=1, keepdims=True)
    w = jnp.where(attrs == m, 128.0 - kf, 0.0)
    idxf = 128.0 - jnp.max(w, axis=1, keepdims=True)
    oh = (kf == idxf).astype(jnp.float32)
    cr4 = cr4_ref[...]
    dn = (((0,), (1,)), ((), ()))
    rows = [
        lax.dot_general(cr4, oh[128 * r:128 * (r + 1), :], dn,
                        preferred_element_type=jnp.float32)
        for r in range(attrs.shape[0] // 128)
    ]
    out_ref[...] = jnp.concatenate(rows, axis=0)


def _soft_kernel(x_ref, r0c_ref, ab_ref, y_ref):
    x4 = x_ref[...]
    cv = r0c_ref[...]
    vi = jax.lax.bitcast_convert_type(cv, jnp.int32)
    r_lo = jax.lax.bitcast_convert_type(vi << 16, jnp.float32)
    r_hi = jax.lax.bitcast_convert_type(vi & jnp.int32(-65536), jnp.float32)
    a = ab_ref[0]
    b = ab_ref[1]

    def f(x, r0):
        u = x / r0
        p = jnp.exp(b * jnp.log(jnp.maximum(u, 1e-30)))
        p = jnp.where(u > 0.0, p, 0.0)
        return x + 0.5 * jnp.tanh(-u - a * p) + 0.5

    tiles = (x4.shape[0], 8, 128)
    y_lo = f(x4[:, 0].reshape(cv.shape), r_lo).reshape(tiles)
    y_hi = f(x4[:, 1].reshape(cv.shape), r_hi).reshape(tiles)
    y_ref[...] = jnp.stack([y_lo, y_hi], axis=1)


def _make_gather_r0(E, TBL):
    FR = 8
    C = FR * 256
    frows = E // 256
    total_chunks = frows // FR
    mesh = plsc.VectorSubcoreMesh(core_axis_name="c", subcore_axis_name="s")

    @functools.partial(
        pl.kernel,
        mesh=mesh,
        compiler_params=pltpu.CompilerParams(needs_layout_passes=False),
        out_type=jax.ShapeDtypeStruct((frows, 128), jnp.float32),
        scratch_types=[
            pltpu.VMEM((TBL,), jnp.float32),
            pltpu.VMEM((2, C), jnp.int32),
            pltpu.VMEM((2, C), jnp.int32),
            pltpu.VMEM((FR, 128), jnp.float32),
            pltpu.VMEM((FR, 128), jnp.float32),
            pltpu.SemaphoreType.DMA,
            pltpu.SemaphoreType.DMA,
            pltpu.SemaphoreType.DMA,
            pltpu.SemaphoreType.DMA,
            pltpu.SemaphoreType.DMA,
        ],
    )
    def gather_r0(rnode_hbm, eidx_hbm, out_hbm, table_v,
                  eb0, eb1, r00, r01, tsem, is0, is1, os0, os1):
        wid = lax.axis_index("s") * _NC + lax.axis_index("c")
        my_chunks = (total_chunks - wid + _NW - 1) // _NW
        ebs, r0s, isems, osems = (eb0, eb1), (r00, r01), (is0, is1), (os0, os1)

        def in_src(ci):
            chunk = wid + ci * _NW
            base = pl.multiple_of(chunk * C, 128)
            return eidx_hbm.at[:, pl.ds(base, C)]

        def out_dst(ci):
            chunk = wid + ci * _NW
            rb = pl.multiple_of(chunk * FR, 8)
            return out_hbm.at[pl.ds(rb, FR), :]

        @pl.when(my_chunks > 0)
        def _():
            pltpu.async_copy(in_src(0), eb0, is0)

        @pl.when(my_chunks > 1)
        def _():
            pltpu.async_copy(in_src(1), eb1, is1)

        pltpu.async_copy(rnode_hbm.at[pl.ds(0, TBL)], table_v, tsem).wait()

        def pair_body(p, carry):
            for b in range(2):
                ci = p * 2 + b
                eb, r0v, isem, osem = ebs[b], r0s[b], isems[b], osems[b]

                @pl.when(ci < my_chunks)
                def _():
                    pltpu.make_async_copy(in_src(ci), eb, isem).wait()

                    @pl.when(ci >= 2)
                    def _():
                        pltpu.make_async_copy(r0v, out_dst(ci - 2), osem).wait()

                    @plsc.parallel_loop(0, FR, unroll=2)
                    def row_body(row):
                        for j in range(8):
                            o = row * 128 + j * _L
                            s_lo = eb[0, pl.ds(o, _L)]
                            s_hi = eb[0, pl.ds(o + 1024, _L)]
                            r_lo = eb[1, pl.ds(o, _L)]
                            r_hi = eb[1, pl.ds(o + 1024, _L)]
                            g_lo = (plsc.load_gather(table_v, [s_lo]) +
                                    plsc.load_gather(table_v, [r_lo]))
                            g_hi = (plsc.load_gather(table_v, [s_hi]) +
                                    plsc.load_gather(table_v, [r_hi]))
                            pk = plsc.bitcast(
                                plsc.pack(g_lo, g_hi,
                                          format=plsc.PackFormat.INTERLEAVED),
                                jnp.float32)
                            r0v[row, pl.ds(j * _L, _L)] = pk

                    pltpu.async_copy(r0v, out_dst(ci), osem)

                    @pl.when(ci + 2 < my_chunks)
                    def _():
                        pltpu.async_copy(in_src(ci + 2), eb, isem)

            return carry

        lax.fori_loop(0, (my_chunks + 1) // 2, pair_body, 0)

        for b in range(2):
            @pl.when(my_chunks > b)
            def _():
                pltpu.make_async_copy(r0s[b], out_dst(b), osems[b]).wait()

    return gather_r0


def kernel(x, node_attrs, edge_index, atomic_numbers, covalent_radii, a, b):
    N, K = node_attrs.shape
    E = x.shape[0]
    BN = 10240
    n_blocks = (N + BN - 1) // BN
    N_pad = n_blocks * BN
    TBL = ((N + 127) // 128 + 7) // 8 * 8 * 128

    azf = atomic_numbers.astype(jnp.float32).reshape(K, 1)
    crp = jnp.zeros((K, 1), jnp.float32)
    crp = crp.at[: covalent_radii.shape[0], 0].set(covalent_radii)

    rnode4 = pl.pallas_call(
        _node_radius_kernel,
        grid=(n_blocks,),
        in_specs=[
            pl.BlockSpec((BN, K), lambda i: (i, 0)),
            pl.BlockSpec((K, 1), lambda i: (0, 0)),
            pl.BlockSpec((K, 1), lambda i: (0, 0)),
        ],
        out_specs=pl.BlockSpec((BN // 128, 128), lambda i: (i, 0)),
        out_shape=jax.ShapeDtypeStruct((N_pad // 128, 128), jnp.float32),
        scratch_shapes=[pltpu.VMEM((128, 1), jnp.float32)],
    )(node_attrs, azf, crp)

    r0c = _make_gather_r0(E, TBL)(rnode4.reshape(N_pad), edge_index)

    W = 128
    frows = E // 256
    Q = E // 2048
    CQ = 125
    ab = jnp.stack([a.astype(jnp.float32), b.astype(jnp.float32)])
    y = pl.pallas_call(
        _soft_kernel,
        grid=(Q // CQ,),
        in_specs=[
            pl.BlockSpec((CQ, 2, 8, W), lambda i: (i, 0, 0, 0)),
            pl.BlockSpec((8 * CQ, W), lambda i: (i, 0)),
            pl.BlockSpec(memory_space=pltpu.SMEM),
        ],
        out_specs=pl.BlockSpec((CQ, 2, 8, W), lambda i: (i, 0, 0, 0)),
        out_shape=jax.ShapeDtypeStruct((Q, 2, 8, W), jnp.float32),
    )(x.reshape(Q, 2, 8, W), r0c, ab)
    return y.reshape(E, 1)


# speedup vs baseline: 27.1771x; 1.0141x over previous
import functools

import jax
import jax.numpy as jnp
from jax import lax
from jax.experimental import pallas as pl
from jax.experimental.pallas import tpu as pltpu
from jax.experimental.pallas import tpu_sc as plsc

_NC = 2
_NS = 16
_NW = _NC * _NS
_L = 16


def _node_radius_kernel(attrs_ref, azf_ref, cr_ref, out_ref, cr4_ref):
    @pl.when(pl.program_id(0) == 0)
    def _():
        z128 = lax.broadcasted_iota(jnp.int32, (128, 128), 1).astype(
            jnp.float32)
        mz = (azf_ref[...] == z128).astype(jnp.float32)
        cr4_ref[...] = 0.25 * jnp.dot(mz, cr_ref[...],
                                      preferred_element_type=jnp.float32)

    attrs = attrs_ref[...]
    kf = lax.broadcasted_iota(jnp.int32, attrs.shape, 1).astype(jnp.float32)
    m = jnp.max(attrs, axis=1, keepdims=True)
    w = jnp.where(attrs == m, 128.0 - kf, 0.0)
    idxf = 128.0 - jnp.max(w, axis=1, keepdims=True)
    oh = (kf == idxf).astype(jnp.float32)
    cr4 = cr4_ref[...]
    dn = (((0,), (1,)), ((), ()))
    rows = [
        lax.dot_general(cr4, oh[128 * r:128 * (r + 1), :], dn,
                        preferred_element_type=jnp.float32)
        for r in range(attrs.shape[0] // 128)
    ]
    out_ref[...] = jnp.concatenate(rows, axis=0)


def _soft_kernel(x_ref, r0c_ref, ab_ref, y_ref):
    x4 = x_ref[...]
    cv = r0c_ref[...]
    vi = jax.lax.bitcast_convert_type(cv, jnp.int32)
    r_lo = jax.lax.bitcast_convert_type(vi << 16, jnp.float32)
    r_hi = jax.lax.bitcast_convert_type(vi & jnp.int32(-65536), jnp.float32)
    a = ab_ref[0]
    b = ab_ref[1]

    def f(x, r0):
        u = x / r0
        p = jnp.exp(b * jnp.log(jnp.maximum(u, 1e-30)))
        p = jnp.where(u > 0.0, p, 0.0)
        return x + 0.5 * jnp.tanh(-u - a * p) + 0.5

    tiles = (x4.shape[0], 8, 128)
    y_lo = f(x4[:, 0].reshape(cv.shape), r_lo).reshape(tiles)
    y_hi = f(x4[:, 1].reshape(cv.shape), r_hi).reshape(tiles)
    y_ref[...] = jnp.stack([y_lo, y_hi], axis=1)


def _make_gather_r0(E, TBL):
    FR = 8
    C = FR * 256
    frows = E // 256
    total_chunks = frows // FR
    mesh = plsc.VectorSubcoreMesh(core_axis_name="c", subcore_axis_name="s")

    @functools.partial(
        pl.kernel,
        mesh=mesh,
        compiler_params=pltpu.CompilerParams(needs_layout_passes=False),
        out_type=jax.ShapeDtypeStruct((frows, 128), jnp.float32),
        scratch_types=[
            pltpu.VMEM((TBL,), jnp.float32),
            pltpu.VMEM((2, C), jnp.int32),
            pltpu.VMEM((2, C), jnp.int32),
            pltpu.VMEM((FR, 128), jnp.float32),
            pltpu.VMEM((FR, 128), jnp.float32),
            pltpu.SemaphoreType.DMA,
            pltpu.SemaphoreType.DMA,
            pltpu.SemaphoreType.DMA,
            pltpu.SemaphoreType.DMA,
            pltpu.SemaphoreType.DMA,
        ],
    )
    def gather_r0(rnode_hbm, eidx_hbm, out_hbm, table_v,
                  eb0, eb1, r00, r01, tsem, is0, is1, os0, os1):
        wid = lax.axis_index("s") * _NC + lax.axis_index("c")
        my_chunks = (total_chunks - wid + _NW - 1) // _NW
        ebs, r0s, isems, osems = (eb0, eb1), (r00, r01), (is0, is1), (os0, os1)

        def in_src(ci):
            chunk = wid + ci * _NW
            base = pl.multiple_of(chunk * C, 128)
            return eidx_hbm.at[:, pl.ds(base, C)]

        def out_dst(ci):
            chunk = wid + ci * _NW
            rb = pl.multiple_of(chunk * FR, 8)
            return out_hbm.at[pl.ds(rb, FR), :]

        @pl.when(my_chunks > 0)
        def _():
            pltpu.async_copy(in_src(0), eb0, is0)

        @pl.when(my_chunks > 1)
        def _():
            pltpu.async_copy(in_src(1), eb1, is1)

        pltpu.async_copy(rnode_hbm.at[pl.ds(0, TBL)], table_v, tsem).wait()

        def pair_body(p, carry):
            for b in range(2):
                ci = p * 2 + b
                eb, r0v, isem, osem = ebs[b], r0s[b], isems[b], osems[b]

                @pl.when(ci < my_chunks)
                def _():
                    pltpu.make_async_copy(in_src(ci), eb, isem).wait()

                    @pl.when(ci >= 2)
                    def _():
                        pltpu.make_async_copy(r0v, out_dst(ci - 2), osem).wait()

                    @plsc.parallel_loop(0, FR * 8, unroll=4)
                    def grp_body(g):
                        o = g * _L
                        s_lo = eb[0, pl.ds(o, _L)]
                        s_hi = eb[0, pl.ds(o + 1024, _L)]
                        r_lo = eb[1, pl.ds(o, _L)]
                        r_hi = eb[1, pl.ds(o + 1024, _L)]
                        g_lo = (plsc.load_gather(table_v, [s_lo]) +
                                plsc.load_gather(table_v, [r_lo]))
                        g_hi = (plsc.load_gather(table_v, [s_hi]) +
                                plsc.load_gather(table_v, [r_hi]))
                        pk = plsc.bitcast(
                            plsc.pack(g_lo, g_hi,
                                      format=plsc.PackFormat.INTERLEAVED),
                            jnp.float32)
                        r0v[g >> 3, pl.ds((g & 7) * _L, _L)] = pk

                    pltpu.async_copy(r0v, out_dst(ci), osem)

                    @pl.when(ci + 2 < my_chunks)
                    def _():
                        pltpu.async_copy(in_src(ci + 2), eb, isem)

            return carry

        lax.fori_loop(0, (my_chunks + 1) // 2, pair_body, 0)

        for b in range(2):
            @pl.when(my_chunks > b)
            def _():
                pltpu.make_async_copy(r0s[b], out_dst(b), osems[b]).wait()

    return gather_r0


def kernel(x, node_attrs, edge_index, atomic_numbers, covalent_radii, a, b):
    N, K = node_attrs.shape
    E = x.shape[0]
    BN = 10240
    n_blocks = (N + BN - 1) // BN
    N_pad = n_blocks * BN
    TBL = ((N + 127) // 128 + 7) // 8 * 8 * 128

    azf = atomic_numbers.astype(jnp.float32).reshape(K, 1)
    crp = jnp.zeros((K, 1), jnp.float32)
    crp = crp.at[: covalent_radii.shape[0], 0].set(covalent_radii)

    rnode4 = pl.pallas_call(
        _node_radius_kernel,
        grid=(n_blocks,),
        in_specs=[
            pl.BlockSpec((BN, K), lambda i: (i, 0)),
            pl.BlockSpec((K, 1), lambda i: (0, 0)),
            pl.BlockSpec((K, 1), lambda i: (0, 0)),
        ],
        out_specs=pl.BlockSpec((BN // 128, 128), lambda i: (i, 0)),
        out_shape=jax.ShapeDtypeStruct((N_pad // 128, 128), jnp.float32),
        scratch_shapes=[pltpu.VMEM((128, 1), jnp.float32)],
    )(node_attrs, azf, crp)

    r0c = _make_gather_r0(E, TBL)(rnode4.reshape(N_pad), edge_index)

    W = 128
    frows = E // 256
    Q = E // 2048
    CQ = 125
    ab = jnp.stack([a.astype(jnp.float32), b.astype(jnp.float32)])
    y = pl.pallas_call(
        _soft_kernel,
        grid=(Q // CQ,),
        in_specs=[
            pl.BlockSpec((CQ, 2, 8, W), lambda i: (i, 0, 0, 0)),
            pl.BlockSpec((8 * CQ, W), lambda i: (i, 0)),
            pl.BlockSpec(memory_space=pltpu.SMEM),
        ],
        out_specs=pl.BlockSpec((CQ, 2, 8, W), lambda i: (i, 0, 0, 0)),
        out_shape=jax.ShapeDtypeStruct((Q, 2, 8, W), jnp.float32),
    )(x.reshape(Q, 2, 8, W), r0c, ab)
    return y.reshape(E, 1)


# speedup vs baseline: 31.5425x; 1.1606x over previous
import functools

import jax
import jax.numpy as jnp
from jax import lax
from jax.experimental import pallas as pl
from jax.experimental.pallas import tpu as pltpu
from jax.experimental.pallas import tpu_sc as plsc

_NC = 2
_NS = 16
_NW = _NC * _NS
_L = 16


def _node_radius_kernel(attrs_ref, azf_ref, cr_ref, out_ref, cr4_ref):
    @pl.when(pl.program_id(0) == 0)
    def _():
        z128 = lax.broadcasted_iota(jnp.int32, (128, 128), 1).astype(
            jnp.float32)
        mz = (azf_ref[...] == z128).astype(jnp.float32)
        cr4_ref[...] = 0.25 * jnp.dot(mz, cr_ref[...],
                                      preferred_element_type=jnp.float32)

    attrs = attrs_ref[...]
    kf = lax.broadcasted_iota(jnp.int32, attrs.shape, 1).astype(jnp.float32)
    m = jnp.max(attrs, axis=1, keepdims=True)
    w = jnp.where(attrs == m, 128.0 - kf, 0.0)
    idxf = 128.0 - jnp.max(w, axis=1, keepdims=True)
    oh = (kf == idxf).astype(jnp.float32)
    cr4 = cr4_ref[...]
    dn = (((0,), (1,)), ((), ()))
    rows = [
        lax.dot_general(cr4, oh[128 * r:128 * (r + 1), :], dn,
                        preferred_element_type=jnp.float32)
        for r in range(attrs.shape[0] // 128)
    ]
    out_ref[...] = jnp.concatenate(rows, axis=0)


def _soft_kernel(x_ref, r0_ref, ab_ref, y_ref):
    x = x_ref[...]
    u = x / r0_ref[...]
    a = ab_ref[0]
    b = ab_ref[1]
    p = jnp.exp(b * jnp.log(jnp.maximum(u, 1e-30)))
    p = jnp.where(u > 0.0, p, 0.0)
    y_ref[...] = x + 0.5 * jnp.tanh(-u - a * p) + 0.5


def _make_gather_r0(E, TBL):
    RB = 40
    C = RB * 128
    rows = E // 128
    total_chunks = rows // RB
    mesh = plsc.VectorSubcoreMesh(core_axis_name="c", subcore_axis_name="s")

    @functools.partial(
        pl.kernel,
        mesh=mesh,
        compiler_params=pltpu.CompilerParams(needs_layout_passes=False),
        out_type=jax.ShapeDtypeStruct((rows, 128), jnp.float32),
        scratch_types=[
            pltpu.VMEM((TBL,), jnp.float32),
            pltpu.VMEM((2, C), jnp.int32),
            pltpu.VMEM((2, C), jnp.int32),
            pltpu.VMEM((RB, 128), jnp.float32),
            pltpu.VMEM((RB, 128), jnp.float32),
            pltpu.SemaphoreType.DMA,
            pltpu.SemaphoreType.DMA,
            pltpu.SemaphoreType.DMA,
            pltpu.SemaphoreType.DMA,
            pltpu.SemaphoreType.DMA,
        ],
    )
    def gather_r0(rnode_hbm, eidx_hbm, out_hbm, table_v,
                  eb0, eb1, r00, r01, tsem, is0, is1, os0, os1):
        wid = lax.axis_index("s") * _NC + lax.axis_index("c")
        my_chunks = (total_chunks - wid + _NW - 1) // _NW
        ebs, r0s, isems, osems = (eb0, eb1), (r00, r01), (is0, is1), (os0, os1)

        def in_src(ci):
            chunk = wid + ci * _NW
            base = pl.multiple_of(chunk * C, 128)
            return eidx_hbm.at[:, pl.ds(base, C)]

        def out_dst(ci):
            chunk = wid + ci * _NW
            rb = pl.multiple_of(chunk * RB, 8)
            return out_hbm.at[pl.ds(rb, RB), :]

        @pl.when(my_chunks > 0)
        def _():
            pltpu.async_copy(in_src(0), eb0, is0)

        @pl.when(my_chunks > 1)
        def _():
            pltpu.async_copy(in_src(1), eb1, is1)

        pltpu.async_copy(rnode_hbm.at[pl.ds(0, TBL)], table_v, tsem).wait()

        def pair_body(p, carry):
            for b in range(2):
                ci = p * 2 + b
                eb, r0v, isem, osem = ebs[b], r0s[b], isems[b], osems[b]

                @pl.when(ci < my_chunks)
                def _():
                    pltpu.make_async_copy(in_src(ci), eb, isem).wait()

                    @pl.when(ci >= 2)
                    def _():
                        pltpu.make_async_copy(r0v, out_dst(ci - 2), osem).wait()

                    @plsc.parallel_loop(0, RB, unroll=8)
                    def row_body(row):
                        for j in range(128 // _L):
                            o = row * 128 + j * _L
                            s16 = eb[0, pl.ds(o, _L)]
                            r16 = eb[1, pl.ds(o, _L)]
                            g = (plsc.load_gather(table_v, [s16]) +
                                 plsc.load_gather(table_v, [r16]))
                            r0v[row, pl.ds(j * _L, _L)] = g

                    pltpu.async_copy(r0v, out_dst(ci), osem)

                    @pl.when(ci + 2 < my_chunks)
                    def _():
                        pltpu.async_copy(in_src(ci + 2), eb, isem)

            return carry

        lax.fori_loop(0, (my_chunks + 1) // 2, pair_body, 0)

        for b in range(2):
            @pl.when(my_chunks > b)
            def _():
                pltpu.make_async_copy(r0s[b], out_dst(b), osems[b]).wait()

    return gather_r0


def kernel(x, node_attrs, edge_index, atomic_numbers, covalent_radii, a, b):
    N, K = node_attrs.shape
    E = x.shape[0]
    BN = 10240
    n_blocks = (N + BN - 1) // BN
    N_pad = n_blocks * BN
    TBL = ((N + 127) // 128 + 7) // 8 * 8 * 128

    azf = atomic_numbers.astype(jnp.float32).reshape(K, 1)
    crp = jnp.zeros((K, 1), jnp.float32)
    crp = crp.at[: covalent_radii.shape[0], 0].set(covalent_radii)

    rnode4 = pl.pallas_call(
        _node_radius_kernel,
        grid=(n_blocks,),
        in_specs=[
            pl.BlockSpec((BN, K), lambda i: (i, 0)),
            pl.BlockSpec((K, 1), lambda i: (0, 0)),
            pl.BlockSpec((K, 1), lambda i: (0, 0)),
        ],
        out_specs=pl.BlockSpec((BN // 128, 128), lambda i: (i, 0)),
        out_shape=jax.ShapeDtypeStruct((N_pad // 128, 128), jnp.float32),
        scratch_shapes=[pltpu.VMEM((128, 1), jnp.float32)],
    )(node_attrs, azf, crp)

    r0 = _make_gather_r0(E, TBL)(rnode4.reshape(N_pad), edge_index)

    W = 128
    rows = E // W
    RB = 5000
    ab = jnp.stack([a.astype(jnp.float32), b.astype(jnp.float32)])
    y = pl.pallas_call(
        _soft_kernel,
        grid=(rows // RB,),
        in_specs=[
            pl.BlockSpec((RB, W), lambda i: (i, 0)),
            pl.BlockSpec((RB, W), lambda i: (i, 0)),
            pl.BlockSpec(memory_space=pltpu.SMEM),
        ],
        out_specs=pl.BlockSpec((RB, W), lambda i: (i, 0)),
        out_shape=jax.ShapeDtypeStruct((rows, W), jnp.float32),
    )(x.reshape(rows, W), r0, ab)
    return y.reshape(E, 1)
